# baseline (speedup 1.0000x reference)
import jax
import jax.numpy as jnp
from jax import lax
from jax.experimental import pallas as pl
from jax.experimental.pallas import tpu as pltpu
from jax.experimental.pallas import tpu_sc as plsc

_B = 16384
_D = 64
_ENT = 1000000
_NC, _NS = 2, 16
_NW = _NC * _NS
_PER_W = _B // _NW
_C = 128
_NCHUNK = _PER_W // _C
_L = 16

_EBLK = 384
_NBLK = (_ENT - 64) // _EBLK
_K1_ITERS = 82


def _rsqrt(x):
    i = plsc.bitcast(x, jnp.int32)
    i = jnp.int32(0x5F3759DF) - lax.shift_right_arithmetic(i, 1)
    y = plsc.bitcast(i, jnp.float32)
    for _ in range(3):
        y = y * (1.5 - 0.5 * x * y * y)
    return y


def _pack_body(entT_hbm, out_hbm, sb0, sb1, ob0, ob1, tbuf,
               sin0, sin1, sout0, sout1):
    wid = lax.axis_index("s") * _NC + lax.axis_index("c")
    iotav = lax.iota(jnp.int32, _L)

    def blk_i0(it):
        b = jnp.minimum(wid + it * _NW, _NBLK - 1)
        return pl.multiple_of(b * _EBLK, _EBLK)

    def fire_in(it, sbuf, sem):
        i0 = blk_i0(it)
        for jhi in range(8):
            pltpu.async_copy(
                entT_hbm.at[pl.ds(jhi * 8, 8), pl.ds(i0, _EBLK)],
                sbuf.at[jhi], sem)

    def wait_in(sbuf, sem):
        for jhi in range(8):
            pltpu.make_async_copy(
                entT_hbm.at[pl.ds(0, 8), pl.ds(0, _EBLK)],
                sbuf.at[jhi], sem).wait()

    def compute(sbuf, obuf):
        @plsc.parallel_loop(0, _EBLK // _L, unroll=8)
        def egrp(g):
            ev = iotav + g * _L
            pv = lax.shift_right_arithmetic(ev, 1)
            cb = (ev & 1) * _D
            for j in range(_D):
                x = sbuf[j >> 3, j & 7, pl.ds(g * _L, _L)]
                cv = cb + ((ev + j) & (_D - 1))
                plsc.store_scatter(obuf, [pv, cv], x)

    def fire_out(it, obuf, sem):
        o0 = pl.multiple_of(blk_i0(it) // 2, _EBLK // 2)
        pltpu.async_copy(obuf, out_hbm.at[pl.ds(o0, _EBLK // 2)], sem)

    def wait_out(obuf, sem):
        pltpu.make_async_copy(
            obuf, out_hbm.at[pl.ds(0, _EBLK // 2)], sem).wait()

    fire_in(0, sb0, sin0)
    fire_in(1, sb1, sin1)

    def loop(k, carry):
        it0 = k * 2
        wait_in(sb0, sin0)

        @pl.when(k > 0)
        def _():
            wait_out(ob0, sout0)
        compute(sb0, ob0)
        fire_out(it0, ob0, sout0)
        fire_in(it0 + 2, sb0, sin0)

        wait_in(sb1, sin1)

        @pl.when(k > 0)
        def _():
            wait_out(ob1, sout1)
        compute(sb1, ob1)
        fire_out(it0 + 1, ob1, sout1)
        fire_in(it0 + 3, sb1, sin1)
        return carry

    lax.fori_loop(0, _K1_ITERS // 2, loop, 0)
    wait_in(sb0, sin0)
    wait_in(sb1, sin1)
    wait_out(ob0, sout0)
    wait_out(ob1, sout1)

    @pl.when(wid == 0)
    def _tail():
        i0 = _NBLK * _EBLK
        tcps = [pltpu.async_copy(
            entT_hbm.at[pl.ds(jhi * 8, 8), pl.ds(i0, 64)],
            tbuf.at[jhi], sin0) for jhi in range(8)]
        for cp in tcps:
            cp.wait()

        def egrp(g, c2):
            ev = iotav + g * _L
            pv = lax.shift_right_arithmetic(ev, 1)
            cb = (ev & 1) * _D
            for j in range(_D):
                x = tbuf[j >> 3, j & 7, pl.ds(g * _L, _L)]
                cv = cb + ((ev + j) & (_D - 1))
                plsc.store_scatter(ob0, [pv, cv], x)
            return c2

        lax.fori_loop(0, 4, egrp, 0)
        pltpu.sync_copy(ob0.at[pl.ds(0, 32)], out_hbm.at[pl.ds(i0 // 2, 32)])


def _score_body(h_hbm, r_hbm, t_hbm, ent2_hbm, rel2_hbm, nrm2_hbm, out_hbm,
                hidx, tidx, ridx, h2, t2, r2, hrow, trow, rrow, nrow, obuf,
                sem):
    wid = lax.axis_index("s") * _NC + lax.axis_index("c")
    iotav = lax.iota(jnp.int32, _L)
    for c in range(_NCHUNK):
        base = wid * _PER_W + c * _C
        pltpu.sync_copy(h_hbm.at[pl.ds(base, _C)], hidx)
        pltpu.sync_copy(t_hbm.at[pl.ds(base, _C)], tidx)
        pltpu.sync_copy(r_hbm.at[pl.ds(base, _C)], ridx)

        def halve(g, carry):
            i16 = pl.ds(g * _L, _L)
            h2[i16] = lax.shift_right_arithmetic(hidx[i16], 1)
            t2[i16] = lax.shift_right_arithmetic(tidx[i16], 1)
            r2[i16] = lax.shift_right_arithmetic(ridx[i16], 1)
            return carry
        lax.fori_loop(0, _C // _L, halve, 0)

        copies = [
            pltpu.async_copy(ent2_hbm.at[h2], hrow, sem),
            pltpu.async_copy(ent2_hbm.at[t2], trow, sem),
            pltpu.async_copy(rel2_hbm.at[r2], rrow, sem),
            pltpu.async_copy(nrm2_hbm.at[r2], nrow, sem),
        ]
        for cp in copies:
            cp.wait()

        def group(g):
            rowv = iotav + g * _L
            hv = hidx[pl.ds(g * _L, _L)]
            tv = tidx[pl.ds(g * _L, _L)]
            rv = ridx[pl.ds(g * _L, _L)]
            hoff = (hv & 1) * _D
            toff = (tv & 1) * _D
            roff = (rv & 1) * _D
            zero = jnp.zeros((_L,), jnp.float32)
            nn, ne, un, uu = zero, zero, zero, zero
            for d in range(_D):
                jv = (iotav + d) & (_D - 1)
                hj = plsc.load_gather(hrow, [rowv, hoff + ((jv + hv) & (_D - 1))])
                tj = plsc.load_gather(trow, [rowv, toff + ((jv + tv) & (_D - 1))])
                nj = plsc.load_gather(nrow, [rowv, roff + jv])
                rj = plsc.load_gather(rrow, [rowv, roff + jv])
                e = hj - tj
                u = e + rj
                nn = nn + nj * nj
                ne = ne + nj * e
                un = un + nj * u
                uu = uu + u * u
            s = nn * _rsqrt(nn)
            a = 1.0 / (s + 1e-12)
            coef = ne * a * a
            dd = uu - 2.0 * coef * un + coef * coef * nn
            dd = jnp.maximum(dd, 0.0)
            obuf[pl.ds(g * _L, _L)] = dd * _rsqrt(dd)

        plsc.parallel_loop(0, _C // _L, unroll=1)(group)
        pltpu.sync_copy(obuf, out_hbm.at[pl.ds(base, _C)])


@jax.jit
def _transh_sc(h, r, t, ent, rel, nrm):
    mesh = plsc.VectorSubcoreMesh(core_axis_name="c", subcore_axis_name="s")
    params = pltpu.CompilerParams(
        needs_layout_passes=False, use_tc_tiling_on_sc=True)

    entT = jnp.swapaxes(ent, 0, 1)
    packed = pl.kernel(
        _pack_body,
        out_type=jax.ShapeDtypeStruct((_ENT // 2, 128), jnp.float32),
        mesh=mesh,
        compiler_params=params,
        scratch_types=[
            pltpu.VMEM((8, 8, _EBLK), jnp.float32),
            pltpu.VMEM((8, 8, _EBLK), jnp.float32),
            pltpu.VMEM((_EBLK // 2, 128), jnp.float32),
            pltpu.VMEM((_EBLK // 2, 128), jnp.float32),
            pltpu.VMEM((8, 8, 64), jnp.float32),
            pltpu.SemaphoreType.DMA,
            pltpu.SemaphoreType.DMA,
            pltpu.SemaphoreType.DMA,
            pltpu.SemaphoreType.DMA,
        ],
    )(entT)

    rel2 = jnp.reshape(rel, (500, 128))
    nrm2 = jnp.reshape(nrm, (500, 128))
    return pl.kernel(
        _score_body,
        out_type=jax.ShapeDtypeStruct((_B,), jnp.float32),
        mesh=mesh,
        compiler_params=params,
        scratch_types=[
            pltpu.VMEM((_C,), jnp.int32),
            pltpu.VMEM((_C,), jnp.int32),
            pltpu.VMEM((_C,), jnp.int32),
            pltpu.VMEM((_C,), jnp.int32),
            pltpu.VMEM((_C,), jnp.int32),
            pltpu.VMEM((_C,), jnp.int32),
            pltpu.VMEM((_C, 128), jnp.float32),
            pltpu.VMEM((_C, 128), jnp.float32),
            pltpu.VMEM((_C, 128), jnp.float32),
            pltpu.VMEM((_C, 128), jnp.float32),
            pltpu.VMEM((_C,), jnp.float32),
            pltpu.SemaphoreType.DMA,
        ],
    )(h, r, t, packed, rel2, nrm2)


def kernel(h, r, t, emb_entity, emb_relation, emb_normal_vec):
    h = h.astype(jnp.int32)
    r = r.astype(jnp.int32)
    t = t.astype(jnp.int32)
    return _transh_sc(h, r, t, emb_entity, emb_relation, emb_normal_vec)


# speedup vs baseline: 1.1191x; 1.1191x over previous
import jax
import jax.numpy as jnp
from jax import lax
from jax.experimental import pallas as pl
from jax.experimental.pallas import tpu as pltpu
from jax.experimental.pallas import tpu_sc as plsc

_B = 16384
_D = 64
_ENT = 1000000
_NC, _NS = 2, 16
_NW = _NC * _NS
_PER_W = _B // _NW
_C = 128
_NCHUNK = _PER_W // _C
_L = 16

_EBLK = 384
_NBLK = (_ENT - 64) // _EBLK
_K1_ITERS = 82


def _rsqrt(x):
    i = plsc.bitcast(x, jnp.int32)
    i = jnp.int32(0x5F3759DF) - lax.shift_right_arithmetic(i, 1)
    y = plsc.bitcast(i, jnp.float32)
    for _ in range(3):
        y = y * (1.5 - 0.5 * x * y * y)
    return y


def _pack_body(entT_hbm, out_hbm, sb0, sb1, ob0, ob1, tbuf,
               sin0, sin1, sout0, sout1):
    wid = lax.axis_index("s") * _NC + lax.axis_index("c")
    iotav = lax.iota(jnp.int32, _L)

    def blk_i0(it):
        b = jnp.minimum(wid + it * _NW, _NBLK - 1)
        return pl.multiple_of(b * _EBLK, _EBLK)

    def fire_in(it, sbuf, sem):
        i0 = blk_i0(it)
        for jhi in range(8):
            pltpu.async_copy(
                entT_hbm.at[pl.ds(jhi * 8, 8), pl.ds(i0, _EBLK)],
                sbuf.at[jhi], sem)

    def wait_in(sbuf, sem):
        for jhi in range(8):
            pltpu.make_async_copy(
                entT_hbm.at[pl.ds(0, 8), pl.ds(0, _EBLK)],
                sbuf.at[jhi], sem).wait()

    def compute(sbuf, obuf):
        @plsc.parallel_loop(0, _EBLK // _L, unroll=4)
        def egrp(g):
            ev = iotav + g * _L
            pv = lax.shift_right_arithmetic(ev, 1)
            cb = (ev & 1) * _D
            for j in range(_D):
                x = sbuf[j >> 3, j & 7, pl.ds(g * _L, _L)]
                cv = cb + ((ev + j) & (_D - 1))
                plsc.store_scatter(obuf, [pv, cv], x)

    def fire_out(it, obuf, sem):
        o0 = pl.multiple_of(blk_i0(it) // 2, _EBLK // 2)
        pltpu.async_copy(obuf, out_hbm.at[pl.ds(o0, _EBLK // 2)], sem)

    def wait_out(obuf, sem):
        pltpu.make_async_copy(
            obuf, out_hbm.at[pl.ds(0, _EBLK // 2)], sem).wait()

    fire_in(0, sb0, sin0)
    fire_in(1, sb1, sin1)

    def loop(k, carry):
        it0 = k * 2
        wait_in(sb0, sin0)

        @pl.when(k > 0)
        def _():
            wait_out(ob0, sout0)
        compute(sb0, ob0)
        fire_out(it0, ob0, sout0)
        fire_in(it0 + 2, sb0, sin0)

        wait_in(sb1, sin1)

        @pl.when(k > 0)
        def _():
            wait_out(ob1, sout1)
        compute(sb1, ob1)
        fire_out(it0 + 1, ob1, sout1)
        fire_in(it0 + 3, sb1, sin1)
        return carry

    lax.fori_loop(0, _K1_ITERS // 2, loop, 0)
    wait_in(sb0, sin0)
    wait_in(sb1, sin1)
    wait_out(ob0, sout0)
    wait_out(ob1, sout1)

    @pl.when(wid == 0)
    def _tail():
        i0 = _NBLK * _EBLK
        tcps = [pltpu.async_copy(
            entT_hbm.at[pl.ds(jhi * 8, 8), pl.ds(i0, 64)],
            tbuf.at[jhi], sin0) for jhi in range(8)]
        for cp in tcps:
            cp.wait()

        def egrp(g, c2):
            ev = iotav + g * _L
            pv = lax.shift_right_arithmetic(ev, 1)
            cb = (ev & 1) * _D
            for j in range(_D):
                x = tbuf[j >> 3, j & 7, pl.ds(g * _L, _L)]
                cv = cb + ((ev + j) & (_D - 1))
                plsc.store_scatter(ob0, [pv, cv], x)
            return c2

        lax.fori_loop(0, 4, egrp, 0)
        pltpu.sync_copy(ob0.at[pl.ds(0, 32)], out_hbm.at[pl.ds(i0 // 2, 32)])


def _score_body(h_hbm, r_hbm, t_hbm, ent2_hbm, rel2_hbm, nrm2_hbm, out_hbm,
                hidxs, tidxs, ridx, h2s, t2s, r2, hrows, trows, rrow, nrow,
                obuf, sem, semhts):
    wid = lax.axis_index("s") * _NC + lax.axis_index("c")
    iotav = lax.iota(jnp.int32, _L)

    def stage(c):
        base = wid * _PER_W + c * _C
        hidx, tidx, h2, t2 = hidxs[c & 1], tidxs[c & 1], h2s[c & 1], t2s[c & 1]
        pltpu.sync_copy(h_hbm.at[pl.ds(base, _C)], hidx)
        pltpu.sync_copy(t_hbm.at[pl.ds(base, _C)], tidx)

        def halve(g, carry):
            i16 = pl.ds(g * _L, _L)
            h2[i16] = lax.shift_right_arithmetic(hidx[i16], 1)
            t2[i16] = lax.shift_right_arithmetic(tidx[i16], 1)
            return carry
        lax.fori_loop(0, _C // _L, halve, 0)
        return [pltpu.async_copy(ent2_hbm.at[h2], hrows[c & 1], semhts[c & 1]),
                pltpu.async_copy(ent2_hbm.at[t2], trows[c & 1], semhts[c & 1])]

    pending = {0: stage(0)}
    for c in range(_NCHUNK):
        base = wid * _PER_W + c * _C
        hidx, tidx = hidxs[c & 1], tidxs[c & 1]
        hrow, trow = hrows[c & 1], trows[c & 1]
        pltpu.sync_copy(r_hbm.at[pl.ds(base, _C)], ridx)

        def halver(g, carry):
            i16 = pl.ds(g * _L, _L)
            r2[i16] = lax.shift_right_arithmetic(ridx[i16], 1)
            return carry
        lax.fori_loop(0, _C // _L, halver, 0)
        rcopies = [
            pltpu.async_copy(rel2_hbm.at[r2], rrow, sem),
            pltpu.async_copy(nrm2_hbm.at[r2], nrow, sem),
        ]
        if c + 1 < _NCHUNK:
            pending[c + 1] = stage(c + 1)
        for cp in pending.pop(c) + rcopies:
            cp.wait()

        def group(g):
            rowv = iotav + g * _L
            hv = hidx[pl.ds(g * _L, _L)]
            tv = tidx[pl.ds(g * _L, _L)]
            rv = ridx[pl.ds(g * _L, _L)]
            hoff = (hv & 1) * _D
            toff = (tv & 1) * _D
            roff = (rv & 1) * _D
            zero = jnp.zeros((_L,), jnp.float32)
            nn, ne, un, uu = zero, zero, zero, zero
            for d in range(_D):
                jv = (iotav + d) & (_D - 1)
                hj = plsc.load_gather(hrow, [rowv, hoff + ((jv + hv) & (_D - 1))])
                tj = plsc.load_gather(trow, [rowv, toff + ((jv + tv) & (_D - 1))])
                nj = plsc.load_gather(nrow, [rowv, roff + jv])
                rj = plsc.load_gather(rrow, [rowv, roff + jv])
                e = hj - tj
                u = e + rj
                nn = nn + nj * nj
                ne = ne + nj * e
                un = un + nj * u
                uu = uu + u * u
            s = nn * _rsqrt(nn)
            a = 1.0 / (s + 1e-12)
            coef = ne * a * a
            dd = uu - 2.0 * coef * un + coef * coef * nn
            dd = jnp.maximum(dd, 0.0)
            obuf[pl.ds(g * _L, _L)] = dd * _rsqrt(dd)

        plsc.parallel_loop(0, _C // _L, unroll=1)(group)
        pltpu.sync_copy(obuf, out_hbm.at[pl.ds(base, _C)])


@jax.jit
def _transh_sc(h, r, t, ent, rel, nrm):
    mesh = plsc.VectorSubcoreMesh(core_axis_name="c", subcore_axis_name="s")
    params = pltpu.CompilerParams(
        needs_layout_passes=False, use_tc_tiling_on_sc=True)

    entT = jnp.swapaxes(ent, 0, 1)
    packed = pl.kernel(
        _pack_body,
        out_type=jax.ShapeDtypeStruct((_ENT // 2, 128), jnp.float32),
        mesh=mesh,
        compiler_params=params,
        scratch_types=[
            pltpu.VMEM((8, 8, _EBLK), jnp.float32),
            pltpu.VMEM((8, 8, _EBLK), jnp.float32),
            pltpu.VMEM((_EBLK // 2, 128), jnp.float32),
            pltpu.VMEM((_EBLK // 2, 128), jnp.float32),
            pltpu.VMEM((8, 8, 64), jnp.float32),
            pltpu.SemaphoreType.DMA,
            pltpu.SemaphoreType.DMA,
            pltpu.SemaphoreType.DMA,
            pltpu.SemaphoreType.DMA,
        ],
    )(entT)

    rel2 = jnp.reshape(rel, (500, 128))
    nrm2 = jnp.reshape(nrm, (500, 128))
    return pl.kernel(
        _score_body,
        out_type=jax.ShapeDtypeStruct((_B,), jnp.float32),
        mesh=mesh,
        compiler_params=params,
        scratch_types=[
            [pltpu.VMEM((_C,), jnp.int32)] * 2,
            [pltpu.VMEM((_C,), jnp.int32)] * 2,
            pltpu.VMEM((_C,), jnp.int32),
            [pltpu.VMEM((_C,), jnp.int32)] * 2,
            [pltpu.VMEM((_C,), jnp.int32)] * 2,
            pltpu.VMEM((_C,), jnp.int32),
            [pltpu.VMEM((_C, 128), jnp.float32)] * 2,
            [pltpu.VMEM((_C, 128), jnp.float32)] * 2,
            pltpu.VMEM((_C, 128), jnp.float32),
            pltpu.VMEM((_C, 128), jnp.float32),
            pltpu.VMEM((_C,), jnp.float32),
            pltpu.SemaphoreType.DMA,
            [pltpu.SemaphoreType.DMA] * 2,
        ],
    )(h, r, t, packed, rel2, nrm2)


def kernel(h, r, t, emb_entity, emb_relation, emb_normal_vec):
    h = h.astype(jnp.int32)
    r = r.astype(jnp.int32)
    t = t.astype(jnp.int32)
    return _transh_sc(h, r, t, emb_entity, emb_relation, emb_normal_vec)
